# baseline (speedup 1.0000x reference)
import jax
import jax.numpy as jnp
from jax import lax
from jax.experimental import pallas as pl
from jax.experimental.pallas import tpu as pltpu

_L = 16
_NKH = 896
_HP = 128
_S = 384
_SA = 512
_NCOL = _L * _NKH + _HP
_V_OUT = 40000
_V_BLK = 1024
_T_PER_CHUNK = 2
_SUB = 512
_N_SUB = _V_BLK // _SUB


def _fused_body(slab_ref, wcombo_ref, wa_ref, b_ref, x_ref, out_ref):
    x = x_ref[...]
    wa = wa_ref[...]
    b = b_ref[...]

    lane = lax.broadcasted_iota(jnp.int32, (_SUB, _SA - _S), 1)
    onecol = jnp.where(lane == 0, 1.0, 0.0).astype(jnp.bfloat16)

    cw = _T_PER_CHUNK * _NKH
    for s in range(_N_SUB):
        slab = slab_ref[s * _SUB:(s + 1) * _SUB, :]
        slab_aug = jnp.concatenate([slab, onecol], axis=1)
        pooled = None
        for c in range(_L // _T_PER_CHUNK):
            p = jnp.dot(slab_aug, wcombo_ref[:, c * cw:(c + 1) * cw],
                        preferred_element_type=jnp.float32)
            for i in range(_T_PER_CHUNK):
                cand = p[:, i * _NKH:(i + 1) * _NKH]
                pooled = cand if pooled is None else jnp.maximum(pooled, cand)

        feat = jnp.tanh(pooled).astype(jnp.bfloat16)
        wproj = jnp.dot(slab_aug, wcombo_ref[:, _L * _NKH:],
                        preferred_element_type=jnp.float32)

        y = jnp.tanh(jnp.dot(feat, wa, preferred_element_type=jnp.float32)
                     + wproj + b)

        out_ref[:, s * _SUB:(s + 1) * _SUB] = lax.dot_general(
            x, y, (((1,), (1,)), ((), ())),
            preferred_element_type=jnp.float32)


def kernel(slab, wcombo, mask, wa, bias, x):
    B = x.shape[0]
    n_blk = -(-_V_OUT // _V_BLK)

    x32 = x.astype(jnp.float32)

    wcombo_aug = jnp.zeros((_SA, _NCOL), jnp.bfloat16)
    wcombo_aug = wcombo_aug.at[:_S, :].set(wcombo)
    wcombo_aug = wcombo_aug.at[_S, :_L * _NKH].set(
        mask.reshape(-1).astype(jnp.bfloat16))

    return pl.pallas_call(
        _fused_body,
        out_shape=jax.ShapeDtypeStruct((B, _V_OUT), jnp.float32),
        grid=(n_blk,),
        in_specs=[
            pl.BlockSpec((_V_BLK, _S), lambda j: (j, 0)),
            pl.BlockSpec((_SA, _NCOL), lambda j: (0, 0)),
            pl.BlockSpec((_NKH, _HP), lambda j: (0, 0)),
            pl.BlockSpec((1, _HP), lambda j: (0, 0)),
            pl.BlockSpec((B, _HP), lambda j: (0, 0)),
        ],
        out_specs=pl.BlockSpec((B, _V_BLK), lambda j: (0, j)),
        compiler_params=pltpu.CompilerParams(
            dimension_semantics=("arbitrary",),
            vmem_limit_bytes=56 * 1024 * 1024),
    )(slab, wcombo_aug, wa, bias, x32)


# speedup vs baseline: 1.0101x; 1.0101x over previous
import jax
import jax.numpy as jnp
from jax import lax
from jax.experimental import pallas as pl
from jax.experimental.pallas import tpu as pltpu

_L = 16
_NKH = 896
_HP = 128
_S = 384
_NCOL = _L * _NKH + _HP
_V_OUT = 40000
_V_BLK = 1024
_T_PER_CHUNK = 2
_SUB = 512
_N_SUB = _V_BLK // _SUB


def _fused_body(slab_ref, wcombo_ref, mask_ref, wa_ref, b_ref, x_ref, out_ref):
    mask = mask_ref[...]
    x = x_ref[...]
    wa = wa_ref[...]
    b = b_ref[...]

    cw = _T_PER_CHUNK * _NKH
    for s in range(_N_SUB):
        slab = slab_ref[s * _SUB:(s + 1) * _SUB, :]
        pooled = None
        for c in range(_L // _T_PER_CHUNK):
            p = jnp.dot(slab, wcombo_ref[:, c * cw:(c + 1) * cw],
                        preferred_element_type=jnp.float32)
            for i in range(_T_PER_CHUNK):
                t = c * _T_PER_CHUNK + i
                cand = p[:, i * _NKH:(i + 1) * _NKH] + mask[t:t + 1, :]
                pooled = cand if pooled is None else jnp.maximum(pooled, cand)

        feat = jnp.tanh(pooled).astype(jnp.bfloat16)
        wproj = jnp.dot(slab, wcombo_ref[:, _L * _NKH:],
                        preferred_element_type=jnp.float32)

        y = jnp.tanh(jnp.dot(feat, wa, preferred_element_type=jnp.float32)
                     + wproj + b)

        out_ref[:, s * _SUB:(s + 1) * _SUB] = lax.dot_general(
            x, y, (((1,), (1,)), ((), ())),
            preferred_element_type=jnp.float32)


def kernel(slab, wcombo, mask, wa, bias, x):
    B = x.shape[0]
    n_blk = -(-_V_OUT // _V_BLK)

    x32 = x.astype(jnp.float32)

    return pl.pallas_call(
        _fused_body,
        out_shape=jax.ShapeDtypeStruct((B, _V_OUT), jnp.float32),
        grid=(n_blk,),
        in_specs=[
            pl.BlockSpec((_V_BLK, _S), lambda j: (j, 0)),
            pl.BlockSpec((_S, _NCOL), lambda j: (0, 0)),
            pl.BlockSpec((_L, _NKH), lambda j: (0, 0)),
            pl.BlockSpec((_NKH, _HP), lambda j: (0, 0)),
            pl.BlockSpec((1, _HP), lambda j: (0, 0)),
            pl.BlockSpec((B, _HP), lambda j: (0, 0)),
        ],
        out_specs=pl.BlockSpec((B, _V_BLK), lambda j: (0, j)),
        compiler_params=pltpu.CompilerParams(
            dimension_semantics=("arbitrary",),
            vmem_limit_bytes=56 * 1024 * 1024),
    )(slab, wcombo, mask, wa, bias, x32)


# speedup vs baseline: 1.0187x; 1.0085x over previous
import jax
import jax.numpy as jnp
from jax import lax
from jax.experimental import pallas as pl
from jax.experimental.pallas import tpu as pltpu

_L = 16
_NKH = 896
_HP = 128
_S = 384
_NCOL = _L * _NKH + _HP
_V_OUT = 40000
_V_BLK = 1024
_T_PER_CHUNK = 2


def _fused_body(slab_ref, wcombo_ref, mask_ref, wa_ref, b_ref, x_ref, out_ref):
    slab = slab_ref[...]
    mask = mask_ref[...]

    cw = _T_PER_CHUNK * _NKH
    pooled = None
    for c in range(_L // _T_PER_CHUNK):
        p = jnp.dot(slab, wcombo_ref[:, c * cw:(c + 1) * cw],
                    preferred_element_type=jnp.float32)
        for i in range(_T_PER_CHUNK):
            t = c * _T_PER_CHUNK + i
            cand = p[:, i * _NKH:(i + 1) * _NKH] + mask[t:t + 1, :]
            pooled = cand if pooled is None else jnp.maximum(pooled, cand)

    feat = jnp.tanh(pooled).astype(jnp.bfloat16)
    wproj = jnp.dot(slab, wcombo_ref[:, _L * _NKH:],
                    preferred_element_type=jnp.float32)

    y = jnp.tanh(jnp.dot(feat, wa_ref[...], preferred_element_type=jnp.float32)
                 + wproj + b_ref[...])

    out_ref[...] = lax.dot_general(x_ref[...], y, (((1,), (1,)), ((), ())),
                                   preferred_element_type=jnp.float32)


def kernel(slab, wcombo, mask, wa, bias, x):
    B = x.shape[0]
    n_blk = -(-_V_OUT // _V_BLK)

    x32 = x.astype(jnp.float32)

    return pl.pallas_call(
        _fused_body,
        out_shape=jax.ShapeDtypeStruct((B, _V_OUT), jnp.float32),
        grid=(n_blk,),
        in_specs=[
            pl.BlockSpec((_V_BLK, _S), lambda j: (j, 0)),
            pl.BlockSpec((_S, _NCOL), lambda j: (0, 0)),
            pl.BlockSpec((_L, _NKH), lambda j: (0, 0)),
            pl.BlockSpec((_NKH, _HP), lambda j: (0, 0)),
            pl.BlockSpec((1, _HP), lambda j: (0, 0)),
            pl.BlockSpec((B, _HP), lambda j: (0, 0)),
        ],
        out_specs=pl.BlockSpec((B, _V_BLK), lambda j: (0, j)),
        compiler_params=pltpu.CompilerParams(
            dimension_semantics=("arbitrary",),
            vmem_limit_bytes=56 * 1024 * 1024),
    )(slab, wcombo, mask, wa, bias, x32)
